# Initial kernel scaffold; baseline (speedup 1.0000x reference)
import jax
import jax.numpy as jnp
from jax.experimental import pallas as pl


def kernel(x, edge_index, W0, b0, L0W, L0b, g0, be0, W1, b1, L1W, L1b, g1, be1):
    raise NotImplementedError("write your pallas kernel here")


# speedup vs baseline: 12.7610x; 12.7610x over previous
import functools

import jax
import jax.numpy as jnp
from jax import lax
from jax.experimental import pallas as pl
from jax.experimental.pallas import tpu as pltpu
from jax.experimental.pallas import tpu_sc as plsc

N = 10000
E = 320000
D = 128

NC = 2
NS = 16
NW = NC * NS
EPW = E // NW
C = 80
NCHUNK = EPW // C
NPAD = 10240
RPT = NPAD // NS
ZR = 80
DPT = NPAD // NS

_mesh = plsc.VectorSubcoreMesh(core_axis_name="c", subcore_axis_name="s")


def _seg_body(ht, src, dst, zrows, out, acc, src_v, dst_v, rows_v, stage, sem):
    c = lax.axis_index("c")
    s = lax.axis_index("s")
    wid = s * NC + c
    pltpu.sync_copy(zrows, stage)
    row0 = s * RPT
    for k in range(RPT // ZR):
        pltpu.sync_copy(stage, acc.at[pl.ds(row0 + k * ZR, ZR)])
    plsc.subcore_barrier()
    base = wid * EPW

    @pl.loop(0, NCHUNK)
    def _(i):
        off = base + i * C
        pltpu.sync_copy(src.at[pl.ds(off, C)], src_v)
        pltpu.sync_copy(dst.at[pl.ds(off, C)], dst_v)
        pltpu.async_copy(ht.at[src_v], rows_v, sem).wait()
        pltpu.sync_copy(rows_v, acc.at[dst_v], add=True)

    plsc.subcore_barrier()

    @pl.when(s < NS - 1)
    def _():
        for k in range(RPT // ZR):
            pltpu.sync_copy(acc.at[pl.ds(row0 + k * ZR, ZR)], stage)
            pltpu.sync_copy(stage, out.at[c, pl.ds(row0 + k * ZR, ZR)])

    @pl.when(s == NS - 1)
    def _():
        for k in range((N - (NS - 1) * RPT) // ZR):
            pltpu.sync_copy(acc.at[pl.ds(row0 + k * ZR, ZR)], stage)
            pltpu.sync_copy(stage, out.at[c, pl.ds(row0 + k * ZR, ZR)])


_sc_segsum = functools.partial(
    pl.kernel,
    out_type=jax.ShapeDtypeStruct((NC, N, D), jnp.float32),
    mesh=_mesh,
    scratch_types=[
        pltpu.VMEM_SHARED((NPAD, D), jnp.float32),
        pltpu.VMEM((C,), jnp.int32),
        pltpu.VMEM((C,), jnp.int32),
        pltpu.VMEM((C, D), jnp.float32),
        pltpu.VMEM((ZR, D), jnp.float32),
        pltpu.SemaphoreType.DMA,
    ],
)(_seg_body)


def _deg_body(dst, out, acc, dst_v, ones_v, stage):
    c = lax.axis_index("c")
    s = lax.axis_index("s")
    wid = s * NC + c
    @pl.loop(0, DPT // 16)
    def _(i):
        stage[pl.ds(i * 16, 16)] = jnp.zeros((16,), jnp.float32)

    for j in range(C // 16):
        ones_v[pl.ds(j * 16, 16)] = jnp.ones((16,), jnp.float32)
    pltpu.sync_copy(stage, acc.at[pl.ds(s * DPT, DPT)])
    plsc.subcore_barrier()
    base = wid * EPW

    @pl.loop(0, NCHUNK)
    def _(i):
        pltpu.sync_copy(dst.at[pl.ds(base + i * C, C)], dst_v)
        pltpu.sync_copy(ones_v, acc.at[dst_v], add=True)

    plsc.subcore_barrier()
    n0 = s * DPT
    ntail = N - (NS - 1) * DPT

    @pl.when(s < NS - 1)
    def _():
        pltpu.sync_copy(acc.at[pl.ds(n0, DPT)], stage)
        pltpu.sync_copy(stage, out.at[pl.ds(c * N + n0, DPT)])

    @pl.when(s == NS - 1)
    def _():
        pltpu.sync_copy(acc.at[pl.ds(n0, ntail)], stage.at[pl.ds(0, ntail)])
        pltpu.sync_copy(stage.at[pl.ds(0, ntail)], out.at[pl.ds(c * N + n0, ntail)])


_sc_degree = functools.partial(
    pl.kernel,
    out_type=jax.ShapeDtypeStruct((NC * N,), jnp.float32),
    mesh=_mesh,
    scratch_types=[
        pltpu.VMEM_SHARED((NPAD,), jnp.float32),
        pltpu.VMEM((C,), jnp.int32),
        pltpu.VMEM((C,), jnp.float32),
        pltpu.VMEM((DPT,), jnp.float32),
    ],
)(_deg_body)


BN = 1000
GRID = N // BN

_row = pl.BlockSpec((BN, D), lambda i: (i, 0))
_col = pl.BlockSpec((BN, 1), lambda i: (i, 0))
_vec = pl.BlockSpec((1, D), lambda i: (0, 0))
_mat = pl.BlockSpec((D, D), lambda i: (0, 0))


def _tc1_body(x, W0, L0W, L0b, degA, degB, dinv_o, ht0_o, r0_o):
    deg = degA[...] + degB[...] + 1.0
    dinv = lax.rsqrt(deg)
    dinv_o[...] = dinv
    xv = x[...]
    h = jnp.dot(xv, W0[...], preferred_element_type=jnp.float32)
    ht0_o[...] = h * dinv
    r0_o[...] = jnp.dot(xv, L0W[...], preferred_element_type=jnp.float32) + L0b[...]


_tc1 = pl.pallas_call(
    _tc1_body,
    grid=(GRID,),
    in_specs=[_row, _mat, _mat, _vec, _col, _col],
    out_specs=[_col, _row, _row],
    out_shape=[
        jax.ShapeDtypeStruct((N, 1), jnp.float32),
        jax.ShapeDtypeStruct((N, D), jnp.float32),
        jax.ShapeDtypeStruct((N, D), jnp.float32),
    ],
)


def _ln_relu(y, g, be):
    mu = jnp.mean(y, axis=-1, keepdims=True)
    yc = y - mu
    var = jnp.mean(yc * yc, axis=-1, keepdims=True)
    return jax.nn.relu(yc * lax.rsqrt(var + 1e-5) * g + be)


def _tc2_body(accA, accB, ht0, dinv, r0, b0, g0, be0, W1, L1W, L1b, ht1_o, r1_o):
    dv = dinv[...]
    y = dv * (accA[...] + accB[...] + ht0[...]) + b0[...] + r0[...]
    h = _ln_relu(y, g0[...], be0[...])
    ht1_o[...] = jnp.dot(h, W1[...], preferred_element_type=jnp.float32) * dv
    r1_o[...] = jnp.dot(h, L1W[...], preferred_element_type=jnp.float32) + L1b[...]


_tc2 = pl.pallas_call(
    _tc2_body,
    grid=(GRID,),
    in_specs=[_row, _row, _row, _col, _row, _vec, _vec, _vec, _mat, _mat, _vec],
    out_specs=[_row, _row],
    out_shape=[
        jax.ShapeDtypeStruct((N, D), jnp.float32),
        jax.ShapeDtypeStruct((N, D), jnp.float32),
    ],
)


def _tc3_body(accA, accB, ht1, dinv, r1, b1, g1, be1, out_o):
    y = dinv[...] * (accA[...] + accB[...] + ht1[...]) + b1[...] + r1[...]
    out_o[...] = _ln_relu(y, g1[...], be1[...])


_tc3 = pl.pallas_call(
    _tc3_body,
    grid=(GRID,),
    in_specs=[_row, _row, _row, _col, _row, _vec, _vec, _vec],
    out_specs=_row,
    out_shape=jax.ShapeDtypeStruct((N, D), jnp.float32),
)


def kernel(x, edge_index, W0, b0, L0W, L0b, g0, be0, W1, b1, L1W, L1b, g1, be1):
    src = edge_index[0]
    dst = edge_index[1]
    zrows = jnp.zeros((ZR, D), jnp.float32)

    degp = _sc_degree(dst)
    degA = degp[:N].reshape(N, 1)
    degB = degp[N:].reshape(N, 1)
    dinv, ht0, r0 = _tc1(x, W0, L0W, L0b.reshape(1, D), degA, degB)

    acc0 = _sc_segsum(ht0, src, dst, zrows)
    ht1, r1 = _tc2(
        acc0[0], acc0[1], ht0, dinv, r0,
        b0.reshape(1, D), g0.reshape(1, D), be0.reshape(1, D),
        W1, L1W, L1b.reshape(1, D),
    )

    acc1 = _sc_segsum(ht1, src, dst, zrows)
    out = _tc3(
        acc1[0], acc1[1], ht1, dinv, r1,
        b1.reshape(1, D), g1.reshape(1, D), be1.reshape(1, D),
    )
    return out
